# baseline (speedup 1.0000x reference)
import functools

import jax
import jax.numpy as jnp
from jax import lax
from jax.experimental import pallas as pl
from jax.experimental.pallas import tpu as pltpu
from jax.experimental.pallas import tpu_sc as plsc

_LANES = 16


def _sc_bias_rows_body(rows_per_w, n_sel, nc, sel_hbm, bias_hbm, out_hbm,
                       idx_v, bias_v, chunk_v):
    wid = lax.axis_index("s") * nc + lax.axis_index("c")
    pltpu.sync_copy(sel_hbm, idx_v)
    pltpu.sync_copy(bias_hbm, bias_v)
    bias_vec = bias_v[...]
    zeros = jnp.zeros((_LANES,), jnp.float32)

    @pl.loop(0, rows_per_w // _LANES, unroll=8)
    def _zero(i):
        chunk_v[pl.ds(i * _LANES, _LANES)] = zeros

    base = wid * rows_per_w

    @pl.loop(0, n_sel // _LANES, unroll=8)
    def _scat(i):
        idx = idx_v[pl.ds(i * _LANES, _LANES)]
        in_slab = (idx >= base) & (idx < base + rows_per_w)
        loc = jnp.where(in_slab, idx - base, 0)
        plsc.store_scatter(chunk_v, [loc], bias_vec, mask=in_slab)

    pltpu.sync_copy(chunk_v, out_hbm.at[pl.ds(base, rows_per_w)])


def _tc_add_body(data_ref, bias_ref, out_ref):
    b = bias_ref[...]
    out_ref[...] = data_ref[...] + lax.broadcast_in_dim(
        b, data_ref.shape, (0,))


@jax.jit
def kernel(data, selection, bias):
    n, d = data.shape
    n_sel = selection.shape[0]
    info = plsc.get_sparse_core_info()
    nc = info.num_cores
    nw = nc * info.num_subcores
    rows_per_w = n // nw

    bias16 = jnp.full((_LANES,), bias, dtype=jnp.float32)
    sc_call = pl.kernel(
        functools.partial(_sc_bias_rows_body, rows_per_w, n_sel, nc),
        out_type=jax.ShapeDtypeStruct((n,), jnp.float32),
        mesh=plsc.VectorSubcoreMesh(core_axis_name="c", subcore_axis_name="s"),
        scratch_types=[
            pltpu.VMEM((n_sel,), jnp.int32),
            pltpu.VMEM((_LANES,), jnp.float32),
            pltpu.VMEM((rows_per_w,), jnp.float32),
        ],
        compiler_params=pltpu.CompilerParams(needs_layout_passes=False),
    )
    bias_rows = sc_call(selection, bias16)

    br = 8192
    bc = d // 2
    out = pl.pallas_call(
        _tc_add_body,
        out_shape=jax.ShapeDtypeStruct((n, d), jnp.float32),
        grid=(n // br, d // bc),
        in_specs=[
            pl.BlockSpec((br, bc), lambda i, j: (i, j)),
            pl.BlockSpec((br,), lambda i, j: (i,)),
        ],
        out_specs=pl.BlockSpec((br, bc), lambda i, j: (i, j)),
    )(data, bias_rows)
    return out


# speedup vs baseline: 1.0502x; 1.0502x over previous
import functools

import jax
import jax.numpy as jnp
from jax import lax
from jax.experimental import pallas as pl
from jax.experimental.pallas import tpu as pltpu
from jax.experimental.pallas import tpu_sc as plsc

_LANES = 16


def _sc_bias_rows_body(rows_per_w, n_sel, nc, sel_hbm, bias_hbm, out_hbm,
                       idx_v, bias_v, chunk_v):
    wid = lax.axis_index("s") * nc + lax.axis_index("c")
    pltpu.sync_copy(sel_hbm, idx_v)
    pltpu.sync_copy(bias_hbm, bias_v)
    bias_vec = bias_v[...]
    zeros = jnp.zeros((_LANES,), jnp.float32)

    @pl.loop(0, rows_per_w // _LANES, unroll=8)
    def _zero(i):
        chunk_v[pl.ds(i * _LANES, _LANES)] = zeros

    base = wid * rows_per_w

    @pl.loop(0, n_sel // _LANES, unroll=8)
    def _scat(i):
        idx = idx_v[pl.ds(i * _LANES, _LANES)]
        in_slab = (idx >= base) & (idx < base + rows_per_w)
        loc = jnp.where(in_slab, idx - base, 0)
        plsc.store_scatter(chunk_v, [loc], bias_vec, mask=in_slab)

    pltpu.sync_copy(chunk_v, out_hbm.at[pl.ds(base, rows_per_w)])


def _tc_add_body(data_ref, bias_ref, out_ref):
    br = data_ref.shape[0]
    b = bias_ref[pl.ds(pl.program_id(0) * br, br)]
    out_ref[...] = data_ref[...] + lax.broadcast_in_dim(
        b, data_ref.shape, (0,))


@jax.jit
def kernel(data, selection, bias):
    n, d = data.shape
    n_sel = selection.shape[0]
    info = plsc.get_sparse_core_info()
    nc = info.num_cores
    nw = nc * info.num_subcores
    rows_per_w = n // nw

    bias16 = jnp.full((_LANES,), bias, dtype=jnp.float32)
    sc_call = pl.kernel(
        functools.partial(_sc_bias_rows_body, rows_per_w, n_sel, nc),
        out_type=jax.ShapeDtypeStruct((n,), jnp.float32),
        mesh=plsc.VectorSubcoreMesh(core_axis_name="c", subcore_axis_name="s"),
        scratch_types=[
            pltpu.VMEM((n_sel,), jnp.int32),
            pltpu.VMEM((_LANES,), jnp.float32),
            pltpu.VMEM((rows_per_w,), jnp.float32),
        ],
        compiler_params=pltpu.CompilerParams(needs_layout_passes=False),
    )
    bias_rows = sc_call(selection, bias16)

    br = 8192
    out = pl.pallas_call(
        _tc_add_body,
        out_shape=jax.ShapeDtypeStruct((n, d), jnp.float32),
        grid=(n // br,),
        in_specs=[
            pl.BlockSpec((br, d), lambda i: (i, 0)),
            pl.BlockSpec((n,), lambda i: (0,)),
        ],
        out_specs=pl.BlockSpec((br, d), lambda i: (i, 0)),
    )(data, bias_rows)
    return out


# speedup vs baseline: 1.0505x; 1.0003x over previous
import functools

import jax
import jax.numpy as jnp
from jax import lax
from jax.experimental import pallas as pl
from jax.experimental.pallas import tpu as pltpu
from jax.experimental.pallas import tpu_sc as plsc

_LANES = 16


def _sc_bias_rows_body(rows_per_w, n_sel, nc, sel_hbm, bias_hbm, out_hbm,
                       idx_v, bias_v, chunk_v):
    wid = lax.axis_index("s") * nc + lax.axis_index("c")
    pltpu.sync_copy(sel_hbm, idx_v)
    pltpu.sync_copy(bias_hbm, bias_v)
    bias_vec = bias_v[...]
    zeros = jnp.zeros((_LANES,), jnp.float32)

    @pl.loop(0, rows_per_w // _LANES, unroll=8)
    def _zero(i):
        chunk_v[pl.ds(i * _LANES, _LANES)] = zeros

    base = wid * rows_per_w

    @pl.loop(0, n_sel // _LANES, unroll=8)
    def _scat(i):
        idx = idx_v[pl.ds(i * _LANES, _LANES)]
        in_slab = (idx >= base) & (idx < base + rows_per_w)
        loc = jnp.where(in_slab, idx - base, 0)
        plsc.store_scatter(chunk_v, [loc], bias_vec, mask=in_slab)

    pltpu.sync_copy(chunk_v, out_hbm.at[pl.ds(base, rows_per_w)])


def _tc_add_body(data_ref, bias_ref, out_ref):
    b = bias_ref[...]
    out_ref[...] = data_ref[...] + lax.broadcast_in_dim(
        b, data_ref.shape, (0,))


@jax.jit
def kernel(data, selection, bias):
    n, d = data.shape
    n_sel = selection.shape[0]
    info = plsc.get_sparse_core_info()
    nc = info.num_cores
    nw = nc * info.num_subcores
    rows_per_w = n // nw

    bias16 = jnp.full((_LANES,), bias, dtype=jnp.float32)
    sc_call = pl.kernel(
        functools.partial(_sc_bias_rows_body, rows_per_w, n_sel, nc),
        out_type=jax.ShapeDtypeStruct((n,), jnp.float32),
        mesh=plsc.VectorSubcoreMesh(core_axis_name="c", subcore_axis_name="s"),
        scratch_types=[
            pltpu.VMEM((n_sel,), jnp.int32),
            pltpu.VMEM((_LANES,), jnp.float32),
            pltpu.VMEM((rows_per_w,), jnp.float32),
        ],
        compiler_params=pltpu.CompilerParams(needs_layout_passes=False),
    )
    bias_rows = sc_call(selection, bias16)

    br = 8192
    out = pl.pallas_call(
        _tc_add_body,
        out_shape=jax.ShapeDtypeStruct((n, d), jnp.float32),
        grid=(n // br,),
        in_specs=[
            pl.BlockSpec((br, d), lambda i: (i, 0)),
            pl.BlockSpec((br,), lambda i: (i,)),
        ],
        out_specs=pl.BlockSpec((br, d), lambda i: (i, 0)),
    )(data, bias_rows)
    return out
